# Initial kernel scaffold; baseline (speedup 1.0000x reference)
import jax
import jax.numpy as jnp
from jax.experimental import pallas as pl


def kernel(input_ids, position_ids, token_embedding, position_embedding):
    raise NotImplementedError("write your pallas kernel here")


# speedup vs baseline: 1.4569x; 1.4569x over previous
import functools

import jax
import jax.numpy as jnp
from jax import lax
from jax.experimental import pallas as pl
from jax.experimental.pallas import tpu as pltpu
from jax.experimental.pallas import tpu_sc as plsc

LANES = 16


def _build_kernel(n_tokens, hidden, chunk, n_workers):
    n_chunks = n_tokens // (n_workers * chunk)
    b_per_w = n_tokens // n_workers
    slices_per_row = hidden // LANES

    mesh = plsc.VectorSubcoreMesh(core_axis_name="c", subcore_axis_name="s")

    @functools.partial(
        pl.kernel,
        mesh=mesh,
        out_type=jax.ShapeDtypeStruct((n_tokens, hidden), jnp.float32),
        scratch_types=[
            pltpu.VMEM((chunk,), jnp.int32),
            pltpu.VMEM((chunk,), jnp.int32),
            pltpu.VMEM((chunk, hidden), jnp.float32),
            pltpu.VMEM((chunk, hidden), jnp.float32),
            pltpu.SemaphoreType.DMA,
            pltpu.SemaphoreType.DMA,
        ],
    )
    def k(tok_ids, pos_ids, tok_table, pos_table, out,
          tok_idx, pos_idx, tok_rows, pos_rows, sem_t, sem_p):
        wid = lax.axis_index("s") * 2 + lax.axis_index("c")
        base = wid * b_per_w

        def chunk_body(ci, _):
            off = base + ci * chunk
            pltpu.sync_copy(tok_ids.at[pl.ds(off, chunk)], tok_idx)
            pltpu.sync_copy(pos_ids.at[pl.ds(off, chunk)], pos_idx)
            ct = pltpu.async_copy(tok_table.at[tok_idx], tok_rows, sem_t)
            cp = pltpu.async_copy(pos_table.at[pos_idx], pos_rows, sem_p)
            ct.wait()
            cp.wait()

            def add_row(r, _):
                for j in range(slices_per_row):
                    sl = pl.ds(j * LANES, LANES)
                    tok_rows[r, sl] = tok_rows[r, sl] + pos_rows[r, sl]
                return _

            lax.fori_loop(0, chunk, add_row, None)
            pltpu.sync_copy(tok_rows, out.at[pl.ds(off, chunk)])
            return _

        lax.fori_loop(0, n_chunks, chunk_body, None)

    return k


def kernel(input_ids, position_ids, token_embedding, position_embedding):
    batch, seq = input_ids.shape
    vocab, hidden = token_embedding.shape
    n_tokens = batch * seq

    n_workers = 32
    chunk = 64
    assert n_tokens % (n_workers * chunk) == 0

    tok_flat = input_ids.reshape(n_tokens).astype(jnp.int32)
    pos_flat = position_ids.reshape(n_tokens).astype(jnp.int32)
    k = _build_kernel(n_tokens, hidden, chunk, n_workers)
    out = k(tok_flat, pos_flat, token_embedding, position_embedding)
    return out.reshape(batch, seq, hidden)
